# baseline (speedup 1.0000x reference)
import functools

import jax
import jax.numpy as jnp
from jax import lax
from jax.experimental import pallas as pl
from jax.experimental.pallas import tpu as pltpu
from jax.experimental.pallas import tpu_sc as plsc

N = 10000
E = 320000
D = 128
L = 128
DR = 64
T = 20
S = 4
MPT = 500

NC = 2
NS = 16
NW = NC * NS
EPW = E // NW
C = 80
NCHUNK = EPW // C
RPS = 624
REM = N - NS * RPS


def _enc_body(x_ref, y_ref, wx_ref, wy_ref, be_ref, wms_ref, wmd_ref, bm_ref,
              h_ref, a_ref, b_ref):
    h = jnp.dot(x_ref[...], wx_ref[...], preferred_element_type=jnp.float32)
    h += jnp.dot(y_ref[...], wy_ref[...], preferred_element_type=jnp.float32)
    h = jnp.maximum(h + be_ref[...], 0.0)
    h_ref[...] = h
    a_ref[...] = jnp.dot(h, wms_ref[...], preferred_element_type=jnp.float32)
    b_ref[...] = (jnp.dot(h, wmd_ref[...], preferred_element_type=jnp.float32)
                  + bm_ref[...])


def _edge_body(a_hbm, b_hbm, idx_hbm, zero_hbm, out_hbm,
               ix0, ix1, sc0, sc1, a0, a1, b0, b1, agg_sh,
               sem_i0, sem_i1, sem_a0, sem_a1, sem_b0, sem_b1,
               sem_s0, sem_s1):
    cid = lax.axis_index("c")
    sid = lax.axis_index("s")
    wid = sid * NC + cid
    pltpu.sync_copy(zero_hbm, agg_sh.at[pl.ds(sid * RPS, RPS)])

    @pl.when(sid == 0)
    def _zero_rem():
        pltpu.sync_copy(zero_hbm.at[pl.ds(0, REM)],
                        agg_sh.at[pl.ds(NS * RPS, REM)])

    plsc.subcore_barrier()

    base0 = wid * NCHUNK
    idx = (ix0, ix1)
    scat = (sc0, sc1)
    abufs = (a0, a1)
    bbufs = (b0, b1)
    sems_i = (sem_i0, sem_i1)
    sems_a = (sem_a0, sem_a1)
    sems_b = (sem_b0, sem_b1)
    sems_s = (sem_s0, sem_s1)

    def issue_idx(i, b):
        pltpu.async_copy(idx_hbm.at[pl.ds((base0 + i) * 2 * C, 2 * C)],
                         idx[b], sems_i[b])

    def wait_idx(i, b):
        pltpu.make_async_copy(idx_hbm.at[pl.ds((base0 + i) * 2 * C, 2 * C)],
                              idx[b], sems_i[b]).wait()

    def issue_rows(b):
        pltpu.async_copy(a_hbm.at[idx[b].at[pl.ds(0, C)]], abufs[b],
                         sems_a[b])
        pltpu.async_copy(b_hbm.at[idx[b].at[pl.ds(C, C)]], bbufs[b],
                         sems_b[b])

    def wait_rows(b):
        pltpu.make_async_copy(a_hbm.at[idx[b].at[pl.ds(0, C)]], abufs[b],
                              sems_a[b]).wait()
        pltpu.make_async_copy(b_hbm.at[idx[b].at[pl.ds(C, C)]], bbufs[b],
                              sems_b[b]).wait()

    def wait_scat(b):
        pltpu.make_async_copy(abufs[b], agg_sh.at[scat[b]], sems_s[b]).wait()

    issue_idx(0, 0)
    issue_idx(1, 1)
    wait_idx(0, 0)
    issue_rows(0)

    def step(i, b):
        wait_rows(b)
        for k in range(C // 16):
            scat[b][pl.ds(k * 16, 16)] = idx[b][pl.ds(C + k * 16, 16)]

        @pl.when(i + 2 < NCHUNK)
        def _pf_idx():
            issue_idx(i + 2, b)

        @pl.when(i + 1 < NCHUNK)
        def _pf_rows():
            @pl.when(i >= 1)
            def _ws():
                wait_scat(1 - b)

            wait_idx(i + 1, 1 - b)
            issue_rows(1 - b)

        def row(jj, c2):
            for k in range(L // 16):
                sl = pl.ds(k * 16, 16)
                abufs[b][jj, sl] = jnp.maximum(
                    abufs[b][jj, sl] + bbufs[b][jj, sl], 0.0)
            return c2

        lax.fori_loop(0, C, row, 0)

        pltpu.async_copy(abufs[b], agg_sh.at[scat[b]], sems_s[b], add=True)

    def pairstep(j, carry):
        step(2 * j, 0)
        step(2 * j + 1, 1)
        return carry

    lax.fori_loop(0, NCHUNK // 2, pairstep, 0)
    step(NCHUNK - 1, 0)
    wait_scat(1)
    wait_scat(0)

    plsc.subcore_barrier()
    pltpu.sync_copy(agg_sh.at[pl.ds(sid * RPS, RPS)],
                    out_hbm.at[pl.ds(cid * N + sid * RPS, RPS)])

    @pl.when(sid == 0)
    def _copy_rem():
        pltpu.sync_copy(agg_sh.at[pl.ds(NS * RPS, REM)],
                        out_hbm.at[pl.ds(cid * N + NS * RPS, REM)])


_edge_kernel = functools.partial(
    pl.kernel,
    out_type=jax.ShapeDtypeStruct((2 * N, L), jnp.float32),
    mesh=plsc.VectorSubcoreMesh(core_axis_name="c", subcore_axis_name="s"),
    scratch_types=[
        pltpu.VMEM((2 * C,), jnp.int32),
        pltpu.VMEM((2 * C,), jnp.int32),
        pltpu.VMEM((C,), jnp.int32),
        pltpu.VMEM((C,), jnp.int32),
        pltpu.VMEM((C, L), jnp.float32),
        pltpu.VMEM((C, L), jnp.float32),
        pltpu.VMEM((C, L), jnp.float32),
        pltpu.VMEM((C, L), jnp.float32),
        pltpu.VMEM_SHARED((N, L), jnp.float32),
        pltpu.SemaphoreType.DMA,
        pltpu.SemaphoreType.DMA,
        pltpu.SemaphoreType.DMA,
        pltpu.SemaphoreType.DMA,
        pltpu.SemaphoreType.DMA,
        pltpu.SemaphoreType.DMA,
        pltpu.SemaphoreType.DMA,
        pltpu.SemaphoreType.DMA,
    ],
)(_edge_body)


def _dec_body(h_ref, g0_ref, g1_ref, wuh_ref, wua_ref, bu_ref, w1_ref, b1_ref,
              w2_ref, b2_ref, rt_ref):
    agg = (g0_ref[...] + g1_ref[...]).reshape(MPT, L)
    h2 = jnp.dot(h_ref[...].reshape(MPT, L), wuh_ref[...],
                 preferred_element_type=jnp.float32)
    h2 += jnp.dot(agg, wua_ref[...], preferred_element_type=jnp.float32)
    h2 = jnp.maximum(h2 + bu_ref[...], 0.0)
    z = jnp.maximum(
        jnp.dot(h2, w1_ref[...], preferred_element_type=jnp.float32)
        + b1_ref[...], 0.0)
    rn = (jnp.dot(z, w2_ref[...], preferred_element_type=jnp.float32)
          + b2_ref[...])
    rt = jnp.sum(rn, axis=0, keepdims=True) * (1.0 / MPT)
    rt_ref[...] = rt.reshape(1, 1, DR)


def _sub_body(m_ref, rt_ref, r_ref):
    r_ref[...] = jnp.dot(m_ref[...], rt_ref[...],
                         preferred_element_type=jnp.float32)


def kernel(x, y, edge_index, subtask_index, W_enc, b_enc, W_msg, b_msg,
           W_upd, b_upd, W1, b1, W2, b2):
    src = edge_index[0].astype(jnp.int32)
    dst = edge_index[1].astype(jnp.int32)
    be = b_enc.reshape(1, L)
    bm = b_msg.reshape(1, L)
    bu = b_upd.reshape(1, L)
    b1r = b1.reshape(1, L)
    b2r = b2.reshape(1, DR)

    RB = 2000
    h, A, B = pl.pallas_call(
        _enc_body,
        grid=(N // RB,),
        in_specs=[
            pl.BlockSpec((RB, D), lambda i: (i, 0)),
            pl.BlockSpec((RB, 3), lambda i: (i, 0)),
            pl.BlockSpec((D, L), lambda i: (0, 0)),
            pl.BlockSpec((3, L), lambda i: (0, 0)),
            pl.BlockSpec((1, L), lambda i: (0, 0)),
            pl.BlockSpec((L, L), lambda i: (0, 0)),
            pl.BlockSpec((L, L), lambda i: (0, 0)),
            pl.BlockSpec((1, L), lambda i: (0, 0)),
        ],
        out_specs=[
            pl.BlockSpec((RB, L), lambda i: (i, 0)),
            pl.BlockSpec((RB, L), lambda i: (i, 0)),
            pl.BlockSpec((RB, L), lambda i: (i, 0)),
        ],
        out_shape=[
            jax.ShapeDtypeStruct((N, L), jnp.float32),
            jax.ShapeDtypeStruct((N, L), jnp.float32),
            jax.ShapeDtypeStruct((N, L), jnp.float32),
        ],
    )(x, y, W_enc[:D], W_enc[D:], be, W_msg[:L], W_msg[L:], bm)

    zero_blk = jnp.zeros((RPS, L), jnp.float32)
    idx_il = jnp.concatenate(
        [src.reshape(NW, NCHUNK, 1, C), dst.reshape(NW, NCHUNK, 1, C)],
        axis=2).reshape(NW * NCHUNK * 2 * C)
    agg2 = _edge_kernel(A, B, idx_il, zero_blk)
    agg2 = agg2.reshape(2 * T, MPT, L)

    rt3 = pl.pallas_call(
        _dec_body,
        grid=(T,),
        in_specs=[
            pl.BlockSpec((1, MPT, L), lambda t: (t, 0, 0)),
            pl.BlockSpec((1, MPT, L), lambda t: (t, 0, 0)),
            pl.BlockSpec((1, MPT, L), lambda t: (t + T, 0, 0)),
            pl.BlockSpec((L, L), lambda t: (0, 0)),
            pl.BlockSpec((L, L), lambda t: (0, 0)),
            pl.BlockSpec((1, L), lambda t: (0, 0)),
            pl.BlockSpec((L, L), lambda t: (0, 0)),
            pl.BlockSpec((1, L), lambda t: (0, 0)),
            pl.BlockSpec((L, DR), lambda t: (0, 0)),
            pl.BlockSpec((1, DR), lambda t: (0, 0)),
        ],
        out_specs=pl.BlockSpec((1, 1, DR), lambda t: (t, 0, 0)),
        out_shape=jax.ShapeDtypeStruct((T, 1, DR), jnp.float32),
    )(h.reshape(T, MPT, L), agg2, agg2, W_upd[:L], W_upd[L:], bu, W1, b1r,
      W2, b2r)
    rt = rt3.reshape(T, DR)

    onehot = (subtask_index[None, :] == jnp.arange(S, dtype=jnp.int32)[:, None])
    Mw = onehot.astype(jnp.float32)
    cnt = Mw.sum(axis=1, keepdims=True)
    Mw = Mw / jnp.maximum(cnt, 1.0)
    Mpad = jnp.zeros((8, T), jnp.float32).at[:S].set(Mw)

    r8 = pl.pallas_call(
        _sub_body,
        in_specs=[
            pl.BlockSpec((8, T), lambda: (0, 0)),
            pl.BlockSpec((T, DR), lambda: (0, 0)),
        ],
        out_specs=pl.BlockSpec((8, DR), lambda: (0, 0)),
        out_shape=jax.ShapeDtypeStruct((8, DR), jnp.float32),
    )(Mpad, rt)
    return r8[:S]


# speedup vs baseline: 1.0639x; 1.0639x over previous
import functools

import jax
import jax.numpy as jnp
from jax import lax
from jax.experimental import pallas as pl
from jax.experimental.pallas import tpu as pltpu
from jax.experimental.pallas import tpu_sc as plsc

N = 10000
E = 320000
D = 128
L = 128
DR = 64
T = 20
S = 4
MPT = 500

NC = 2
NS = 16
NW = NC * NS
EPW = E // NW
C = 80
NCHUNK = EPW // C
RPS = 624
REM = N - NS * RPS


def _enc_body(x_ref, y_ref, wx_ref, wy_ref, be_ref, wms_ref, wmd_ref, bm_ref,
              h_ref, a_ref, b_ref):
    h = jnp.dot(x_ref[...], wx_ref[...], preferred_element_type=jnp.float32)
    h += jnp.dot(y_ref[...], wy_ref[...], preferred_element_type=jnp.float32)
    h = jnp.maximum(h + be_ref[...], 0.0)
    h_ref[...] = h
    a_ref[...] = jnp.dot(h, wms_ref[...], preferred_element_type=jnp.float32)
    b_ref[...] = (jnp.dot(h, wmd_ref[...], preferred_element_type=jnp.float32)
                  + bm_ref[...])


def _edge_body(a_hbm, b_hbm, src_hbm, dst_hbm, zero_hbm, out_hbm,
               si0, si1, di0, di1, sc0, sc1, a0, a1, b0, b1, agg_sh,
               sem_i0, sem_i1, sem_a0, sem_a1, sem_b0, sem_b1):
    cid = lax.axis_index("c")
    sid = lax.axis_index("s")
    wid = sid * NC + cid
    pltpu.sync_copy(zero_hbm, agg_sh.at[pl.ds(sid * RPS, RPS)])

    @pl.when(sid == 0)
    def _zero_rem():
        pltpu.sync_copy(zero_hbm.at[pl.ds(0, REM)],
                        agg_sh.at[pl.ds(NS * RPS, REM)])

    plsc.subcore_barrier()

    base0 = wid * EPW
    sidx = (si0, si1)
    didx = (di0, di1)
    scat = (sc0, sc1)
    abufs = (a0, a1)
    bbufs = (b0, b1)
    sems_i = (sem_i0, sem_i1)
    sems_a = (sem_a0, sem_a1)
    sems_b = (sem_b0, sem_b1)

    def issue_idx(i, b):
        pltpu.async_copy(src_hbm.at[pl.ds(base0 + i * C, C)], sidx[b],
                         sems_i[b])
        pltpu.async_copy(dst_hbm.at[pl.ds(base0 + i * C, C)], didx[b],
                         sems_i[b])

    def wait_idx(i, b):
        pltpu.make_async_copy(src_hbm.at[pl.ds(base0 + i * C, C)], sidx[b],
                              sems_i[b]).wait()
        pltpu.make_async_copy(dst_hbm.at[pl.ds(base0 + i * C, C)], didx[b],
                              sems_i[b]).wait()

    def issue_rows(b):
        pltpu.async_copy(a_hbm.at[sidx[b]], abufs[b], sems_a[b])
        pltpu.async_copy(b_hbm.at[didx[b]], bbufs[b], sems_b[b])

    def wait_rows(b):
        pltpu.make_async_copy(a_hbm.at[sidx[b]], abufs[b], sems_a[b]).wait()
        pltpu.make_async_copy(b_hbm.at[didx[b]], bbufs[b], sems_b[b]).wait()

    issue_idx(0, 0)
    issue_idx(1, 1)
    wait_idx(0, 0)
    issue_rows(0)

    def step(i, b):
        wait_rows(b)
        for k in range(C // 16):
            sl = pl.ds(k * 16, 16)
            scat[b][sl] = didx[b][sl]

        @pl.when(i + 2 < NCHUNK)
        def _pf_idx():
            issue_idx(i + 2, b)

        @pl.when(i + 1 < NCHUNK)
        def _pf_rows():
            wait_idx(i + 1, 1 - b)
            issue_rows(1 - b)

        def row(jj, c2):
            for k in range(L // 16):
                sl = pl.ds(k * 16, 16)
                abufs[b][jj, sl] = jnp.maximum(
                    abufs[b][jj, sl] + bbufs[b][jj, sl], 0.0)
            return c2

        lax.fori_loop(0, C, row, 0)
        pltpu.sync_copy(abufs[b], agg_sh.at[scat[b]], add=True)

    def pairstep(j, carry):
        step(2 * j, 0)
        step(2 * j + 1, 1)
        return carry

    lax.fori_loop(0, NCHUNK // 2, pairstep, 0)
    step(NCHUNK - 1, 0)

    plsc.subcore_barrier()
    pltpu.sync_copy(agg_sh.at[pl.ds(sid * RPS, RPS)],
                    out_hbm.at[pl.ds(cid * N + sid * RPS, RPS)])

    @pl.when(sid == 0)
    def _copy_rem():
        pltpu.sync_copy(agg_sh.at[pl.ds(NS * RPS, REM)],
                        out_hbm.at[pl.ds(cid * N + NS * RPS, REM)])


_edge_kernel = functools.partial(
    pl.kernel,
    out_type=jax.ShapeDtypeStruct((2 * N, L), jnp.float32),
    mesh=plsc.VectorSubcoreMesh(core_axis_name="c", subcore_axis_name="s"),
    scratch_types=[
        pltpu.VMEM((C,), jnp.int32),
        pltpu.VMEM((C,), jnp.int32),
        pltpu.VMEM((C,), jnp.int32),
        pltpu.VMEM((C,), jnp.int32),
        pltpu.VMEM((C,), jnp.int32),
        pltpu.VMEM((C,), jnp.int32),
        pltpu.VMEM((C, L), jnp.float32),
        pltpu.VMEM((C, L), jnp.float32),
        pltpu.VMEM((C, L), jnp.float32),
        pltpu.VMEM((C, L), jnp.float32),
        pltpu.VMEM_SHARED((N, L), jnp.float32),
        pltpu.SemaphoreType.DMA,
        pltpu.SemaphoreType.DMA,
        pltpu.SemaphoreType.DMA,
        pltpu.SemaphoreType.DMA,
        pltpu.SemaphoreType.DMA,
        pltpu.SemaphoreType.DMA,
    ],
)(_edge_body)


def _dec_body(h_ref, g0_ref, g1_ref, wuh_ref, wua_ref, bu_ref, w1_ref, b1_ref,
              w2_ref, b2_ref, m3_ref, r_ref):
    agg = (g0_ref[...] + g1_ref[...]).reshape(MPT, L)
    h2 = jnp.dot(h_ref[...].reshape(MPT, L), wuh_ref[...],
                 preferred_element_type=jnp.float32)
    h2 += jnp.dot(agg, wua_ref[...], preferred_element_type=jnp.float32)
    h2 = jnp.maximum(h2 + bu_ref[...], 0.0)
    z = jnp.maximum(
        jnp.dot(h2, w1_ref[...], preferred_element_type=jnp.float32)
        + b1_ref[...], 0.0)
    rn = (jnp.dot(z, w2_ref[...], preferred_element_type=jnp.float32)
          + b2_ref[...])
    rt = jnp.sum(rn, axis=0, keepdims=True) * (1.0 / MPT)

    @pl.when(pl.program_id(0) == 0)
    def _init():
        r_ref[...] = jnp.zeros_like(r_ref)

    r_ref[...] += m3_ref[...].reshape(8, DR) * rt


def kernel(x, y, edge_index, subtask_index, W_enc, b_enc, W_msg, b_msg,
           W_upd, b_upd, W1, b1, W2, b2):
    src = edge_index[0].astype(jnp.int32)
    dst = edge_index[1].astype(jnp.int32)
    be = b_enc.reshape(1, L)
    bm = b_msg.reshape(1, L)
    bu = b_upd.reshape(1, L)
    b1r = b1.reshape(1, L)
    b2r = b2.reshape(1, DR)

    RB = 2000
    h, A, B = pl.pallas_call(
        _enc_body,
        grid=(N // RB,),
        in_specs=[
            pl.BlockSpec((RB, D), lambda i: (i, 0)),
            pl.BlockSpec((RB, 3), lambda i: (i, 0)),
            pl.BlockSpec((D, L), lambda i: (0, 0)),
            pl.BlockSpec((3, L), lambda i: (0, 0)),
            pl.BlockSpec((1, L), lambda i: (0, 0)),
            pl.BlockSpec((L, L), lambda i: (0, 0)),
            pl.BlockSpec((L, L), lambda i: (0, 0)),
            pl.BlockSpec((1, L), lambda i: (0, 0)),
        ],
        out_specs=[
            pl.BlockSpec((RB, L), lambda i: (i, 0)),
            pl.BlockSpec((RB, L), lambda i: (i, 0)),
            pl.BlockSpec((RB, L), lambda i: (i, 0)),
        ],
        out_shape=[
            jax.ShapeDtypeStruct((N, L), jnp.float32),
            jax.ShapeDtypeStruct((N, L), jnp.float32),
            jax.ShapeDtypeStruct((N, L), jnp.float32),
        ],
    )(x, y, W_enc[:D], W_enc[D:], be, W_msg[:L], W_msg[L:], bm)

    zero_blk = jnp.zeros((RPS, L), jnp.float32)
    agg2 = _edge_kernel(A, B, src, dst, zero_blk)
    agg2 = agg2.reshape(2 * T, MPT, L)

    onehot = (subtask_index[None, :] == jnp.arange(S, dtype=jnp.int32)[:, None])
    Mw = onehot.astype(jnp.float32)
    cnt = Mw.sum(axis=1, keepdims=True)
    Mw = Mw / jnp.maximum(cnt, 1.0)
    Mpad = jnp.zeros((8, T), jnp.float32).at[:S].set(Mw)
    m3 = jnp.broadcast_to(Mpad.T[:, :, None], (T, 8, DR))

    r8 = pl.pallas_call(
        _dec_body,
        grid=(T,),
        in_specs=[
            pl.BlockSpec((1, MPT, L), lambda t: (t, 0, 0)),
            pl.BlockSpec((1, MPT, L), lambda t: (t, 0, 0)),
            pl.BlockSpec((1, MPT, L), lambda t: (t + T, 0, 0)),
            pl.BlockSpec((L, L), lambda t: (0, 0)),
            pl.BlockSpec((L, L), lambda t: (0, 0)),
            pl.BlockSpec((1, L), lambda t: (0, 0)),
            pl.BlockSpec((L, L), lambda t: (0, 0)),
            pl.BlockSpec((1, L), lambda t: (0, 0)),
            pl.BlockSpec((L, DR), lambda t: (0, 0)),
            pl.BlockSpec((1, DR), lambda t: (0, 0)),
            pl.BlockSpec((1, 8, DR), lambda t: (t, 0, 0)),
        ],
        out_specs=pl.BlockSpec((8, DR), lambda t: (0, 0)),
        out_shape=jax.ShapeDtypeStruct((8, DR), jnp.float32),
    )(h.reshape(T, MPT, L), agg2, agg2, W_upd[:L], W_upd[L:], bu, W1, b1r,
      W2, b2r, m3)
    return r8[:S]
